# baseline (speedup 1.0000x reference)
import jax
import jax.numpy as jnp
from jax import lax
from jax.experimental import pallas as pl
from jax.experimental.pallas import tpu as pltpu

_N = 4194304
_GRID = 1
_BLOCK = _N // _GRID


def _body(c_ref, w_ref, o_ref):
    o_ref[...] = w_ref[...]

    @pl.when(pl.program_id(0) == 0)
    def _patch():
        a = c_ref[9] / (c_ref[10] * 400000.0) * 0.001
        b = c_ref[11] / c_ref[10]
        head = w_ref[pl.ds(0, 128)]
        idx = lax.broadcasted_iota(jnp.int32, head.shape, 0)
        head = jnp.where(idx == 0, a, head)
        head = jnp.where(idx == 1, b, head)
        o_ref[pl.ds(0, 128)] = head


def kernel(y, w, c, t):
    return pl.pallas_call(
        _body,
        grid=(_GRID,),
        in_specs=[
            pl.BlockSpec(memory_space=pltpu.SMEM),
            pl.BlockSpec((_BLOCK,), lambda i: (i,)),
        ],
        out_specs=pl.BlockSpec((_BLOCK,), lambda i: (i,)),
        out_shape=jax.ShapeDtypeStruct((_N,), jnp.float32),
    )(c, w)


# speedup vs baseline: 1.0476x; 1.0476x over previous
import jax
import jax.numpy as jnp
from jax import lax
from jax.experimental import pallas as pl
from jax.experimental.pallas import tpu as pltpu

_N = 4194304
_K = 8
_CHUNK = _N // _K


def _body(c_ref, w_ref, o_ref, buf, *sems):
    in_sems = sems[:_K]
    out_sems = sems[_K:]
    in_cps = []
    for i in range(_K):
        cp = pltpu.make_async_copy(
            w_ref.at[pl.ds(i * _CHUNK, _CHUNK)], buf.at[i], in_sems[i]
        )
        cp.start()
        in_cps.append(cp)

    a = c_ref[9] / (c_ref[10] * 400000.0) * 0.001
    b = c_ref[11] / c_ref[10]

    out_cps = []
    for i in range(_K):
        in_cps[i].wait()
        if i == 0:
            head = buf[0, pl.ds(0, 128)]
            idx = lax.broadcasted_iota(jnp.int32, head.shape, 0)
            head = jnp.where(idx == 0, a, head)
            head = jnp.where(idx == 1, b, head)
            buf[0, pl.ds(0, 128)] = head
        cp = pltpu.make_async_copy(
            buf.at[i], o_ref.at[pl.ds(i * _CHUNK, _CHUNK)], out_sems[i]
        )
        cp.start()
        out_cps.append(cp)
    for cp in out_cps:
        cp.wait()


def kernel(y, w, c, t):
    return pl.pallas_call(
        _body,
        in_specs=[
            pl.BlockSpec(memory_space=pltpu.SMEM),
            pl.BlockSpec(memory_space=pl.ANY),
        ],
        out_specs=pl.BlockSpec(memory_space=pl.ANY),
        out_shape=jax.ShapeDtypeStruct((_N,), jnp.float32),
        scratch_shapes=[pltpu.VMEM((_K, _CHUNK), jnp.float32)]
        + [pltpu.SemaphoreType.DMA] * (2 * _K),
    )(c, w)


# speedup vs baseline: 1.0672x; 1.0187x over previous
import jax
import jax.numpy as jnp
from jax import lax
from jax.experimental import pallas as pl
from jax.experimental.pallas import tpu as pltpu

_N = 4194304
_K = 8
_CHUNK = _N // _K


def _body(c_ref, w_ref, o_ref, *rest):
    bufs = rest[:_K]
    in_sems = rest[_K:2 * _K]
    out_sems = rest[2 * _K:]
    in_cps = []
    for i in range(_K):
        cp = pltpu.make_async_copy(
            w_ref.at[pl.ds(i * _CHUNK, _CHUNK)], bufs[i], in_sems[i]
        )
        cp.start()
        in_cps.append(cp)

    a = c_ref[9] / (c_ref[10] * 400000.0) * 0.001
    b = c_ref[11] / c_ref[10]

    out_cps = []
    for i in range(_K):
        in_cps[i].wait()
        if i == 0:
            head = bufs[0][pl.ds(0, 128)]
            idx = lax.broadcasted_iota(jnp.int32, head.shape, 0)
            head = jnp.where(idx == 0, a, head)
            head = jnp.where(idx == 1, b, head)
            bufs[0][pl.ds(0, 128)] = head
        cp = pltpu.make_async_copy(
            bufs[i], o_ref.at[pl.ds(i * _CHUNK, _CHUNK)], out_sems[i]
        )
        cp.start()
        out_cps.append(cp)
    for cp in out_cps:
        cp.wait()


def kernel(y, w, c, t):
    return pl.pallas_call(
        _body,
        in_specs=[
            pl.BlockSpec(memory_space=pltpu.SMEM),
            pl.BlockSpec(memory_space=pl.ANY),
        ],
        out_specs=pl.BlockSpec(memory_space=pl.ANY),
        out_shape=jax.ShapeDtypeStruct((_N,), jnp.float32),
        scratch_shapes=[pltpu.VMEM((_CHUNK,), jnp.float32)] * _K
        + [pltpu.SemaphoreType.DMA] * (2 * _K),
    )(c, w)


# speedup vs baseline: 1.0686x; 1.0014x over previous
import jax
import jax.numpy as jnp
from jax import lax
from jax.experimental import pallas as pl
from jax.experimental.pallas import tpu as pltpu

_N = 4194304
_K = 4
_CHUNK = _N // _K


def _body(c_ref, w_ref, o_ref, *rest):
    bufs = rest[:_K]
    in_sems = rest[_K:2 * _K]
    out_sems = rest[2 * _K:]
    in_cps = []
    for i in range(_K):
        cp = pltpu.make_async_copy(
            w_ref.at[pl.ds(i * _CHUNK, _CHUNK)], bufs[i], in_sems[i]
        )
        cp.start()
        in_cps.append(cp)

    a = c_ref[9] / (c_ref[10] * 400000.0) * 0.001
    b = c_ref[11] / c_ref[10]

    out_cps = []
    for i in range(_K):
        in_cps[i].wait()
        if i == 0:
            head = bufs[0][pl.ds(0, 128)]
            idx = lax.broadcasted_iota(jnp.int32, head.shape, 0)
            head = jnp.where(idx == 0, a, head)
            head = jnp.where(idx == 1, b, head)
            bufs[0][pl.ds(0, 128)] = head
        cp = pltpu.make_async_copy(
            bufs[i], o_ref.at[pl.ds(i * _CHUNK, _CHUNK)], out_sems[i]
        )
        cp.start()
        out_cps.append(cp)
    for cp in out_cps:
        cp.wait()


def kernel(y, w, c, t):
    return pl.pallas_call(
        _body,
        in_specs=[
            pl.BlockSpec(memory_space=pltpu.SMEM),
            pl.BlockSpec(memory_space=pl.ANY),
        ],
        out_specs=pl.BlockSpec(memory_space=pl.ANY),
        out_shape=jax.ShapeDtypeStruct((_N,), jnp.float32),
        scratch_shapes=[pltpu.VMEM((_CHUNK,), jnp.float32)] * _K
        + [pltpu.SemaphoreType.DMA] * (2 * _K),
    )(c, w)


# speedup vs baseline: 1.1677x; 1.0927x over previous
import jax
import jax.numpy as jnp
from jax import lax
from jax.experimental import pallas as pl
from jax.experimental.pallas import tpu as pltpu

_N = 4194304
_GRID = 2
_BLOCK = _N // _GRID


def _body(c_ref, w_ref, o_ref):
    o_ref[...] = w_ref[...]

    @pl.when(pl.program_id(0) == 0)
    def _patch():
        a = c_ref[9] / (c_ref[10] * 400000.0) * 0.001
        b = c_ref[11] / c_ref[10]
        head = w_ref[pl.ds(0, 128)]
        idx = lax.broadcasted_iota(jnp.int32, head.shape, 0)
        head = jnp.where(idx == 0, a, head)
        head = jnp.where(idx == 1, b, head)
        o_ref[pl.ds(0, 128)] = head


def kernel(y, w, c, t):
    return pl.pallas_call(
        _body,
        grid=(_GRID,),
        in_specs=[
            pl.BlockSpec(memory_space=pltpu.SMEM),
            pl.BlockSpec((_BLOCK,), lambda i: (i,)),
        ],
        out_specs=pl.BlockSpec((_BLOCK,), lambda i: (i,)),
        out_shape=jax.ShapeDtypeStruct((_N,), jnp.float32),
    )(c, w)
